# Initial kernel scaffold; baseline (speedup 1.0000x reference)
import jax
import jax.numpy as jnp
from jax.experimental import pallas as pl


def kernel(x, edge_index, W, b, att_pre_low, att_pre_band, att_channel_low, att_channel_band, W_mlp, b_mlp):
    raise NotImplementedError("write your pallas kernel here")


# speedup vs baseline: 8.8106x; 8.8106x over previous
import functools

import jax
import jax.numpy as jnp
from jax import lax
from jax.experimental import pallas as pl
from jax.experimental.pallas import tpu as pltpu
from jax.experimental.pallas import tpu_sc as plsc

N_NODES = 10000
N_EDGES = 160000
D = 256
HD = 128

NC = 2
NS = 16
IDXW = 128
EPAD = 163840
ROWS_SUB = EPAD // (NS * IDXW)
ACC_N = 10112
ZROWS = ACC_N // NS
DRAIN = 624
DTAIL = N_NODES - NS * DRAIN

BLK = 400
GRID = N_NODES // BLK


def _sc_mesh():
    return plsc.VectorSubcoreMesh(core_axis_name="c", subcore_axis_name="s")


def _hop_body(u_hbm, gidx_hbm, cidx_hbm, zeros_hbm, v_hbm,
              acc, gi_v, ci_v, rows_v, sem):
    c = lax.axis_index("c")
    s = lax.axis_index("s")
    pltpu.sync_copy(zeros_hbm, acc.at[pl.ds(s * ZROWS, ZROWS)])
    plsc.subcore_barrier()
    pltpu.sync_copy(gidx_hbm.at[c, s], gi_v)
    pltpu.sync_copy(cidx_hbm.at[s], ci_v)

    @pl.loop(0, ROWS_SUB)
    def _(j):
        pltpu.async_copy(u_hbm.at[gi_v.at[j]], rows_v, sem).wait()
        pltpu.sync_copy(rows_v, acc.at[ci_v.at[j]], add=True)

    plsc.subcore_barrier()
    pltpu.sync_copy(acc.at[pl.ds(s * DRAIN, DRAIN)],
                    v_hbm.at[c, pl.ds(s * DRAIN, DRAIN)])

    @pl.when(s == 0)
    def _():
        pltpu.sync_copy(acc.at[pl.ds(NS * DRAIN, DTAIL)],
                        v_hbm.at[c, pl.ds(NS * DRAIN, DTAIL)])


def _sc_hop(u_flat, gidx, cidx, zeros128):
    kern = pl.kernel(
        _hop_body,
        out_type=jax.ShapeDtypeStruct((NC, N_NODES, HD), jnp.float32),
        mesh=_sc_mesh(),
        scratch_types=[
            pltpu.VMEM_SHARED((ACC_N, HD), jnp.float32),
            pltpu.VMEM((ROWS_SUB, IDXW), jnp.int32),
            pltpu.VMEM((ROWS_SUB, IDXW), jnp.int32),
            pltpu.VMEM((IDXW, HD), jnp.float32),
            pltpu.SemaphoreType.DMA,
        ],
    )
    return kern(u_flat, gidx, cidx, zeros128)


def _prep_body(degp_ref, x_ref, u0_ref, dd_ref, rinv_ref):
    deg = degp_ref[0, :, 0:1] + 1.0
    dinv = lax.rsqrt(deg)
    dd_ref[...] = jnp.broadcast_to(1.0 / deg, dd_ref.shape)
    rinv_ref[...] = jnp.broadcast_to(jnp.sqrt(deg), rinv_ref.shape)
    x = x_ref[...]
    u0_ref[0] = x[:, :HD] * dinv
    u0_ref[1] = x[:, HD:] * dinv


def _tc_prep(deg2, x):
    return pl.pallas_call(
        _prep_body,
        grid=(GRID,),
        in_specs=[
            pl.BlockSpec((NC, BLK, HD), lambda i: (0, i, 0)),
            pl.BlockSpec((BLK, D), lambda i: (i, 0)),
        ],
        out_specs=[
            pl.BlockSpec((NC, BLK, HD), lambda i: (0, i, 0)),
            pl.BlockSpec((BLK, 8), lambda i: (i, 0)),
            pl.BlockSpec((BLK, 8), lambda i: (i, 0)),
        ],
        out_shape=[
            jax.ShapeDtypeStruct((NC, N_NODES, HD), jnp.float32),
            jax.ShapeDtypeStruct((N_NODES, 8), jnp.float32),
            jax.ShapeDtypeStruct((N_NODES, 8), jnp.float32),
        ],
    )(deg2, x)


def _glue_body(v_ref, u_ref, dd_ref, un_ref):
    dd = dd_ref[:, 0:1][None]
    un_ref[...] = dd * (v_ref[...] + u_ref[...])


def _tc_glue(v, u, dd):
    return pl.pallas_call(
        _glue_body,
        grid=(GRID,),
        in_specs=[
            pl.BlockSpec((NC, BLK, HD), lambda i: (0, i, 0)),
            pl.BlockSpec((NC, BLK, HD), lambda i: (0, i, 0)),
            pl.BlockSpec((BLK, 8), lambda i: (i, 0)),
        ],
        out_specs=pl.BlockSpec((NC, BLK, HD), lambda i: (0, i, 0)),
        out_shape=jax.ShapeDtypeStruct((NC, N_NODES, HD), jnp.float32),
    )(v, u, dd)


def _elu(z):
    return jnp.where(z > 0, z, jnp.exp(jnp.minimum(z, 0.0)) - 1.0)


def _dot(a, b):
    return jnp.dot(a, b, preferred_element_type=jnp.float32)


def _dense_body(u0_ref, u1_ref, u2_ref, u4_ref, rinv_ref,
                W_ref, b_ref, apl_ref, apb_ref, acl_ref, acb_ref,
                Wm_ref, bm_ref, out_ref):
    rinv = rinv_ref[:, 0:1]

    def xk(u_ref):
        return jnp.concatenate([u_ref[0], u_ref[1]], axis=-1) * rinv

    x0 = xk(u0_ref)
    x1 = xk(u1_ref)
    x2 = xk(u2_ref)
    x4 = xk(u4_ref)
    chans = (x0, x1, x2, x4, x0 - x1, x1 - x2, x2 - x4)
    feats = [_elu(_dot(chans[i], W_ref[i]) + b_ref[i]) for i in range(7)]

    def att(pre_feat, stack_feats, a_pre_ref, a_chan_ref):
        e_pre = _dot(pre_feat, a_pre_ref[...])
        es = [_dot(f, a_chan_ref[...]) + e_pre for f in stack_feats]
        m = jnp.maximum(jnp.maximum(es[0], es[1]), es[2])
        ws = [jnp.exp(e - m) for e in es]
        tot = ws[0] + ws[1] + ws[2]
        acc = ws[0] * stack_feats[0]
        acc += ws[1] * stack_feats[1]
        acc += ws[2] * stack_feats[2]
        return acc / tot

    x_low = att(feats[0], feats[1:4], apl_ref, acl_ref)
    x_band = att(feats[4], feats[4:7], apb_ref, acb_ref)
    out_ref[...] = (_dot(x_low, Wm_ref[0]) + _dot(x_band, Wm_ref[1])
                    + bm_ref[...])


def _tc_dense(u0, u1, u2, u4, rinv, W, b, apl, apb, acl, acb, Wm, bm):
    uspec = pl.BlockSpec((NC, BLK, HD), lambda i: (0, i, 0))
    full = lambda *shape: pl.BlockSpec(shape, lambda i: (0,) * len(shape))
    return pl.pallas_call(
        _dense_body,
        grid=(GRID,),
        in_specs=[
            uspec, uspec, uspec, uspec,
            pl.BlockSpec((BLK, 8), lambda i: (i, 0)),
            full(7, D, D),
            full(7, 1, D),
            full(D, D),
            full(D, D),
            full(D, D),
            full(D, D),
            full(2, D, D),
            full(1, D),
        ],
        out_specs=pl.BlockSpec((BLK, D), lambda i: (i, 0)),
        out_shape=jax.ShapeDtypeStruct((N_NODES, D), jnp.float32),
    )(u0, u1, u2, u4, rinv, W, b, apl, apb, acl, acb, Wm, bm)


def kernel(x, edge_index, W, b, att_pre_low, att_pre_band,
           att_channel_low, att_channel_band, W_mlp, b_mlp):
    row = edge_index[0].astype(jnp.int32)
    col = edge_index[1].astype(jnp.int32)

    npad = EPAD - N_EDGES
    pad_ids = jnp.arange(npad, dtype=jnp.int32)
    rowp = jnp.concatenate([row, (pad_ids * 13) % N_NODES])
    colp = jnp.concatenate([col, N_NODES + (pad_ids % 64)])

    gidx = jnp.stack([rowp, rowp + N_NODES]).reshape(NC, NS, ROWS_SUB, IDXW)
    cidx = colp.reshape(NS, ROWS_SUB, IDXW)

    zeros128 = jnp.zeros((ZROWS, HD), jnp.float32)

    ones_tab = jnp.ones((NC * N_NODES, HD), jnp.float32)
    deg2 = _sc_hop(ones_tab, gidx, cidx, zeros128)
    u0, dd, rinv = _tc_prep(deg2, x)

    us = [u0]
    u = u0
    for _ in range(4):
        v = _sc_hop(u.reshape(NC * N_NODES, HD), gidx, cidx, zeros128)
        u = _tc_glue(v, u, dd)
        us.append(u)

    Wr = W.reshape(7, D, D)
    br = b.reshape(7, 1, D)
    return _tc_dense(
        us[0], us[1], us[2], us[4], rinv,
        Wr, br,
        att_pre_low.reshape(D, D), att_pre_band.reshape(D, D),
        att_channel_low.reshape(D, D), att_channel_band.reshape(D, D),
        W_mlp.reshape(2, D, D), b_mlp.reshape(1, D),
    )


# speedup vs baseline: 10.8680x; 1.2335x over previous
import functools

import jax
import jax.numpy as jnp
from jax import lax
from jax.experimental import pallas as pl
from jax.experimental.pallas import tpu as pltpu
from jax.experimental.pallas import tpu_sc as plsc

N_NODES = 10000
N_EDGES = 160000
D = 256
HD = 128

NC = 2
NS = 16
IDXW = 64
EPAD = 163840
NPH = 2
ROWS_PH = 80
ROWS_SUB = NPH * ROWS_PH
ACC_N = 10112
ZROWS = ACC_N // NS
DRAIN = 624
DTAIL = N_NODES - NS * DRAIN

BLK = 400
GRID = N_NODES // BLK


def _sc_mesh():
    return plsc.VectorSubcoreMesh(core_axis_name="c", subcore_axis_name="s")


def _hop_body(u_hbm, gidx_hbm, cidx_hbm, zeros_hbm, v_hbm,
              acc, gi_v, ci_v, rows0, rows1, sem0, sem1):
    c = lax.axis_index("c")
    s = lax.axis_index("s")
    pltpu.sync_copy(zeros_hbm, acc.at[pl.ds(s * ZROWS, ZROWS)])
    plsc.subcore_barrier()
    for p in range(NPH):
        pltpu.sync_copy(gidx_hbm.at[c, s, p], gi_v)
        pltpu.sync_copy(cidx_hbm.at[s, p], ci_v)

        pltpu.async_copy(u_hbm.at[gi_v.at[0]], rows0, sem0)

        @pl.loop(0, ROWS_PH // 2)
        def _(jj):
            j = jj * 2
            pltpu.async_copy(u_hbm.at[gi_v.at[j + 1]], rows1, sem1)
            pltpu.make_async_copy(u_hbm.at[gi_v.at[j]], rows0, sem0).wait()
            pltpu.sync_copy(rows0, acc.at[ci_v.at[j]], add=True)

            @pl.when(jj < ROWS_PH // 2 - 1)
            def _():
                pltpu.async_copy(u_hbm.at[gi_v.at[j + 2]], rows0, sem0)

            pltpu.make_async_copy(u_hbm.at[gi_v.at[j + 1]], rows1, sem1).wait()
            pltpu.sync_copy(rows1, acc.at[ci_v.at[j + 1]], add=True)

    plsc.subcore_barrier()
    pltpu.sync_copy(acc.at[pl.ds(s * DRAIN, DRAIN)],
                    v_hbm.at[c, pl.ds(s * DRAIN, DRAIN)])

    @pl.when(s == 0)
    def _():
        pltpu.sync_copy(acc.at[pl.ds(NS * DRAIN, DTAIL)],
                        v_hbm.at[c, pl.ds(NS * DRAIN, DTAIL)])


def _sc_hop(u_flat, gidx, cidx, zeros128):
    kern = pl.kernel(
        _hop_body,
        out_type=jax.ShapeDtypeStruct((NC, N_NODES, HD), jnp.float32),
        mesh=_sc_mesh(),
        scratch_types=[
            pltpu.VMEM_SHARED((ACC_N, HD), jnp.float32),
            pltpu.VMEM((ROWS_PH, IDXW), jnp.int32),
            pltpu.VMEM((ROWS_PH, IDXW), jnp.int32),
            pltpu.VMEM((IDXW, HD), jnp.float32),
            pltpu.VMEM((IDXW, HD), jnp.float32),
            pltpu.SemaphoreType.DMA,
            pltpu.SemaphoreType.DMA,
        ],
    )
    return kern(u_flat, gidx, cidx, zeros128)


def _prep_body(degp_ref, x_ref, u0_ref, dd_ref, rinv_ref):
    deg = degp_ref[0, :, 0:1] + 1.0
    dinv = lax.rsqrt(deg)
    dd_ref[...] = jnp.broadcast_to(1.0 / deg, dd_ref.shape)
    rinv_ref[...] = jnp.broadcast_to(jnp.sqrt(deg), rinv_ref.shape)
    x = x_ref[...]
    u0_ref[0] = x[:, :HD] * dinv
    u0_ref[1] = x[:, HD:] * dinv


def _tc_prep(deg2, x):
    return pl.pallas_call(
        _prep_body,
        grid=(GRID,),
        in_specs=[
            pl.BlockSpec((NC, BLK, HD), lambda i: (0, i, 0)),
            pl.BlockSpec((BLK, D), lambda i: (i, 0)),
        ],
        out_specs=[
            pl.BlockSpec((NC, BLK, HD), lambda i: (0, i, 0)),
            pl.BlockSpec((BLK, 8), lambda i: (i, 0)),
            pl.BlockSpec((BLK, 8), lambda i: (i, 0)),
        ],
        out_shape=[
            jax.ShapeDtypeStruct((NC, N_NODES, HD), jnp.float32),
            jax.ShapeDtypeStruct((N_NODES, 8), jnp.float32),
            jax.ShapeDtypeStruct((N_NODES, 8), jnp.float32),
        ],
    )(deg2, x)


def _glue_body(v_ref, u_ref, dd_ref, un_ref):
    dd = dd_ref[:, 0:1][None]
    un_ref[...] = dd * (v_ref[...] + u_ref[...])


def _tc_glue(v, u, dd):
    return pl.pallas_call(
        _glue_body,
        grid=(GRID,),
        in_specs=[
            pl.BlockSpec((NC, BLK, HD), lambda i: (0, i, 0)),
            pl.BlockSpec((NC, BLK, HD), lambda i: (0, i, 0)),
            pl.BlockSpec((BLK, 8), lambda i: (i, 0)),
        ],
        out_specs=pl.BlockSpec((NC, BLK, HD), lambda i: (0, i, 0)),
        out_shape=jax.ShapeDtypeStruct((NC, N_NODES, HD), jnp.float32),
    )(v, u, dd)


def _elu(z):
    return jnp.where(z > 0, z, jnp.exp(jnp.minimum(z, 0.0)) - 1.0)


def _dot(a, b):
    return jnp.dot(a, b, preferred_element_type=jnp.float32)


def _dense_body(u0_ref, u1_ref, u2_ref, u4_ref, rinv_ref,
                W_ref, b_ref, apl_ref, apb_ref, acl_ref, acb_ref,
                Wm_ref, bm_ref, out_ref):
    rinv = rinv_ref[:, 0:1]

    def xk(u_ref):
        return jnp.concatenate([u_ref[0], u_ref[1]], axis=-1) * rinv

    x0 = xk(u0_ref)
    x1 = xk(u1_ref)
    x2 = xk(u2_ref)
    x4 = xk(u4_ref)
    chans = (x0, x1, x2, x4, x0 - x1, x1 - x2, x2 - x4)
    feats = [_elu(_dot(chans[i], W_ref[i]) + b_ref[i]) for i in range(7)]

    def att(pre_feat, stack_feats, a_pre_ref, a_chan_ref):
        e_pre = _dot(pre_feat, a_pre_ref[...])
        es = [_dot(f, a_chan_ref[...]) + e_pre for f in stack_feats]
        m = jnp.maximum(jnp.maximum(es[0], es[1]), es[2])
        ws = [jnp.exp(e - m) for e in es]
        tot = ws[0] + ws[1] + ws[2]
        acc = ws[0] * stack_feats[0]
        acc += ws[1] * stack_feats[1]
        acc += ws[2] * stack_feats[2]
        return acc / tot

    x_low = att(feats[0], feats[1:4], apl_ref, acl_ref)
    x_band = att(feats[4], feats[4:7], apb_ref, acb_ref)
    out_ref[...] = (_dot(x_low, Wm_ref[0]) + _dot(x_band, Wm_ref[1])
                    + bm_ref[...])


def _tc_dense(u0, u1, u2, u4, rinv, W, b, apl, apb, acl, acb, Wm, bm):
    uspec = pl.BlockSpec((NC, BLK, HD), lambda i: (0, i, 0))
    full = lambda *shape: pl.BlockSpec(shape, lambda i: (0,) * len(shape))
    return pl.pallas_call(
        _dense_body,
        grid=(GRID,),
        in_specs=[
            uspec, uspec, uspec, uspec,
            pl.BlockSpec((BLK, 8), lambda i: (i, 0)),
            full(7, D, D),
            full(7, 1, D),
            full(D, D),
            full(D, D),
            full(D, D),
            full(D, D),
            full(2, D, D),
            full(1, D),
        ],
        out_specs=pl.BlockSpec((BLK, D), lambda i: (i, 0)),
        out_shape=jax.ShapeDtypeStruct((N_NODES, D), jnp.float32),
    )(u0, u1, u2, u4, rinv, W, b, apl, apb, acl, acb, Wm, bm)


def kernel(x, edge_index, W, b, att_pre_low, att_pre_band,
           att_channel_low, att_channel_band, W_mlp, b_mlp):
    row = edge_index[0].astype(jnp.int32)
    col = edge_index[1].astype(jnp.int32)

    npad = EPAD - N_EDGES
    pad_ids = jnp.arange(npad, dtype=jnp.int32)
    rowp = jnp.concatenate([row, (pad_ids * 13) % N_NODES])
    colp = jnp.concatenate([col, N_NODES + (pad_ids % 64)])

    gidx = jnp.stack([rowp, rowp + N_NODES]).reshape(NC, NS, NPH, ROWS_PH,
                                                     IDXW)
    cidx = colp.reshape(NS, NPH, ROWS_PH, IDXW)

    zeros128 = jnp.zeros((ZROWS, HD), jnp.float32)

    ones_tab = jnp.ones((NC * N_NODES, HD), jnp.float32)
    deg2 = _sc_hop(ones_tab, gidx, cidx, zeros128)
    u0, dd, rinv = _tc_prep(deg2, x)

    us = [u0]
    u = u0
    for _ in range(4):
        v = _sc_hop(u.reshape(NC * N_NODES, HD), gidx, cidx, zeros128)
        u = _tc_glue(v, u, dd)
        us.append(u)

    Wr = W.reshape(7, D, D)
    br = b.reshape(7, 1, D)
    return _tc_dense(
        us[0], us[1], us[2], us[4], rinv,
        Wr, br,
        att_pre_low.reshape(D, D), att_pre_band.reshape(D, D),
        att_channel_low.reshape(D, D), att_channel_band.reshape(D, D),
        W_mlp.reshape(2, D, D), b_mlp.reshape(1, D),
    )


# speedup vs baseline: 12.2843x; 1.1303x over previous
import functools

import jax
import jax.numpy as jnp
from jax import lax
from jax.experimental import pallas as pl
from jax.experimental.pallas import tpu as pltpu
from jax.experimental.pallas import tpu_sc as plsc

N_NODES = 10000
N_EDGES = 160000
D = 256
HD = 128

NC = 2
NS = 16
IDXW = 64
EPAD = 163840
NPH = 2
ROWS_PH = 80
ROWS_SUB = NPH * ROWS_PH
ACC_N = 10112
ZROWS = ACC_N // NS
DRAIN = 624
DTAIL = N_NODES - NS * DRAIN

BLK = 400
GRID = N_NODES // BLK


def _sc_mesh():
    return plsc.VectorSubcoreMesh(core_axis_name="c", subcore_axis_name="s")


def _deg_body(cidx_hbm, ones_hbm, zeros_hbm, deg_hbm, acc, ci_v, ones_v, sem):
    c = lax.axis_index("c")
    s = lax.axis_index("s")
    pltpu.sync_copy(zeros_hbm, acc.at[pl.ds(s * ZROWS, ZROWS)])
    plsc.subcore_barrier()
    pltpu.sync_copy(ones_hbm, ones_v)
    pltpu.sync_copy(cidx_hbm.at[s, c], ci_v)

    @pl.loop(0, ROWS_PH)
    def _(j):
        pltpu.sync_copy(ones_v, acc.at[ci_v.at[j]], add=True)

    plsc.subcore_barrier()
    pltpu.sync_copy(acc.at[pl.ds(s * DRAIN, DRAIN)],
                    deg_hbm.at[c, pl.ds(s * DRAIN, DRAIN)])

    @pl.when(s == 0)
    def _():
        pltpu.sync_copy(acc.at[pl.ds(NS * DRAIN, DTAIL)],
                        deg_hbm.at[c, pl.ds(NS * DRAIN, DTAIL)])


def _sc_degree(cidx, ones64, zeros128):
    kern = pl.kernel(
        _deg_body,
        out_type=jax.ShapeDtypeStruct((NC, N_NODES, HD), jnp.float32),
        mesh=_sc_mesh(),
        scratch_types=[
            pltpu.VMEM_SHARED((ACC_N, HD), jnp.float32),
            pltpu.VMEM((ROWS_PH, IDXW), jnp.int32),
            pltpu.VMEM((IDXW, HD), jnp.float32),
            pltpu.SemaphoreType.DMA,
        ],
    )
    return kern(cidx, ones64, zeros128)


def _hop_body(u_hbm, gidx_hbm, cidx_hbm, zeros_hbm, v_hbm,
              acc, gi_v, ci_v, rows0, rows1, sem0, sem1):
    c = lax.axis_index("c")
    s = lax.axis_index("s")
    pltpu.sync_copy(zeros_hbm, acc.at[pl.ds(s * ZROWS, ZROWS)])
    plsc.subcore_barrier()
    for p in range(NPH):
        pltpu.sync_copy(gidx_hbm.at[c, s, p], gi_v)
        pltpu.sync_copy(cidx_hbm.at[s, p], ci_v)

        pltpu.async_copy(u_hbm.at[gi_v.at[0]], rows0, sem0)

        @pl.loop(0, ROWS_PH // 2)
        def _(jj):
            j = jj * 2
            pltpu.async_copy(u_hbm.at[gi_v.at[j + 1]], rows1, sem1)
            pltpu.make_async_copy(u_hbm.at[gi_v.at[j]], rows0, sem0).wait()
            pltpu.sync_copy(rows0, acc.at[ci_v.at[j]], add=True)

            @pl.when(jj < ROWS_PH // 2 - 1)
            def _():
                pltpu.async_copy(u_hbm.at[gi_v.at[j + 2]], rows0, sem0)

            pltpu.make_async_copy(u_hbm.at[gi_v.at[j + 1]], rows1, sem1).wait()
            pltpu.sync_copy(rows1, acc.at[ci_v.at[j + 1]], add=True)

    plsc.subcore_barrier()
    pltpu.sync_copy(acc.at[pl.ds(s * DRAIN, DRAIN)],
                    v_hbm.at[c, pl.ds(s * DRAIN, DRAIN)])

    @pl.when(s == 0)
    def _():
        pltpu.sync_copy(acc.at[pl.ds(NS * DRAIN, DTAIL)],
                        v_hbm.at[c, pl.ds(NS * DRAIN, DTAIL)])


def _sc_hop(u_flat, gidx, cidx, zeros128):
    kern = pl.kernel(
        _hop_body,
        out_type=jax.ShapeDtypeStruct((NC, N_NODES, HD), jnp.float32),
        mesh=_sc_mesh(),
        scratch_types=[
            pltpu.VMEM_SHARED((ACC_N, HD), jnp.float32),
            pltpu.VMEM((ROWS_PH, IDXW), jnp.int32),
            pltpu.VMEM((ROWS_PH, IDXW), jnp.int32),
            pltpu.VMEM((IDXW, HD), jnp.float32),
            pltpu.VMEM((IDXW, HD), jnp.float32),
            pltpu.SemaphoreType.DMA,
            pltpu.SemaphoreType.DMA,
        ],
    )
    return kern(u_flat, gidx, cidx, zeros128)


def _prep_body(degp_ref, x_ref, u0_ref, dd_ref, rinv_ref):
    deg = degp_ref[0, :, 0:1] + degp_ref[1, :, 0:1] + 1.0
    dinv = lax.rsqrt(deg)
    dd_ref[...] = jnp.broadcast_to(1.0 / deg, dd_ref.shape)
    rinv_ref[...] = jnp.broadcast_to(jnp.sqrt(deg), rinv_ref.shape)
    x = x_ref[...]
    u0_ref[0] = x[:, :HD] * dinv
    u0_ref[1] = x[:, HD:] * dinv


def _tc_prep(deg2, x):
    return pl.pallas_call(
        _prep_body,
        grid=(GRID,),
        in_specs=[
            pl.BlockSpec((NC, BLK, HD), lambda i: (0, i, 0)),
            pl.BlockSpec((BLK, D), lambda i: (i, 0)),
        ],
        out_specs=[
            pl.BlockSpec((NC, BLK, HD), lambda i: (0, i, 0)),
            pl.BlockSpec((BLK, 8), lambda i: (i, 0)),
            pl.BlockSpec((BLK, 8), lambda i: (i, 0)),
        ],
        out_shape=[
            jax.ShapeDtypeStruct((NC, N_NODES, HD), jnp.float32),
            jax.ShapeDtypeStruct((N_NODES, 8), jnp.float32),
            jax.ShapeDtypeStruct((N_NODES, 8), jnp.float32),
        ],
    )(deg2, x)


def _glue_body(v_ref, u_ref, dd_ref, un_ref):
    dd = dd_ref[:, 0:1][None]
    un_ref[...] = dd * (v_ref[...] + u_ref[...])


def _tc_glue(v, u, dd):
    return pl.pallas_call(
        _glue_body,
        grid=(GRID,),
        in_specs=[
            pl.BlockSpec((NC, BLK, HD), lambda i: (0, i, 0)),
            pl.BlockSpec((NC, BLK, HD), lambda i: (0, i, 0)),
            pl.BlockSpec((BLK, 8), lambda i: (i, 0)),
        ],
        out_specs=pl.BlockSpec((NC, BLK, HD), lambda i: (0, i, 0)),
        out_shape=jax.ShapeDtypeStruct((NC, N_NODES, HD), jnp.float32),
    )(v, u, dd)


def _elu(z):
    return jnp.where(z > 0, z, jnp.exp(jnp.minimum(z, 0.0)) - 1.0)


def _dot(a, b):
    return jnp.dot(a, b, preferred_element_type=jnp.float32)


def _dense_body(u0_ref, u1_ref, u2_ref, u4_ref, rinv_ref,
                W_ref, b_ref, apl_ref, apb_ref, acl_ref, acb_ref,
                Wm_ref, bm_ref, out_ref):
    rinv = rinv_ref[:, 0:1]

    def xk(u_ref):
        return jnp.concatenate([u_ref[0], u_ref[1]], axis=-1) * rinv

    x0 = xk(u0_ref)
    x1 = xk(u1_ref)
    x2 = xk(u2_ref)
    x4 = xk(u4_ref)
    chans = (x0, x1, x2, x4, x0 - x1, x1 - x2, x2 - x4)
    feats = [_elu(_dot(chans[i], W_ref[i]) + b_ref[i]) for i in range(7)]

    def att(pre_feat, stack_feats, a_pre_ref, a_chan_ref):
        e_pre = _dot(pre_feat, a_pre_ref[...])
        es = [_dot(f, a_chan_ref[...]) + e_pre for f in stack_feats]
        m = jnp.maximum(jnp.maximum(es[0], es[1]), es[2])
        ws = [jnp.exp(e - m) for e in es]
        tot = ws[0] + ws[1] + ws[2]
        acc = ws[0] * stack_feats[0]
        acc += ws[1] * stack_feats[1]
        acc += ws[2] * stack_feats[2]
        return acc / tot

    x_low = att(feats[0], feats[1:4], apl_ref, acl_ref)
    x_band = att(feats[4], feats[4:7], apb_ref, acb_ref)
    out_ref[...] = (_dot(x_low, Wm_ref[0]) + _dot(x_band, Wm_ref[1])
                    + bm_ref[...])


def _tc_dense(u0, u1, u2, u4, rinv, W, b, apl, apb, acl, acb, Wm, bm):
    uspec = pl.BlockSpec((NC, BLK, HD), lambda i: (0, i, 0))
    full = lambda *shape: pl.BlockSpec(shape, lambda i: (0,) * len(shape))
    return pl.pallas_call(
        _dense_body,
        grid=(GRID,),
        in_specs=[
            uspec, uspec, uspec, uspec,
            pl.BlockSpec((BLK, 8), lambda i: (i, 0)),
            full(7, D, D),
            full(7, 1, D),
            full(D, D),
            full(D, D),
            full(D, D),
            full(D, D),
            full(2, D, D),
            full(1, D),
        ],
        out_specs=pl.BlockSpec((BLK, D), lambda i: (i, 0)),
        out_shape=jax.ShapeDtypeStruct((N_NODES, D), jnp.float32),
    )(u0, u1, u2, u4, rinv, W, b, apl, apb, acl, acb, Wm, bm)


def kernel(x, edge_index, W, b, att_pre_low, att_pre_band,
           att_channel_low, att_channel_band, W_mlp, b_mlp):
    row = edge_index[0].astype(jnp.int32)
    col = edge_index[1].astype(jnp.int32)

    npad = EPAD - N_EDGES
    pad_ids = jnp.arange(npad, dtype=jnp.int32)
    rowp = jnp.concatenate([row, (pad_ids * 13) % N_NODES])
    colp = jnp.concatenate([col, N_NODES + (pad_ids % 64)])

    gidx = jnp.stack([rowp, rowp + N_NODES]).reshape(NC, NS, NPH, ROWS_PH,
                                                     IDXW)
    cidx = colp.reshape(NS, NPH, ROWS_PH, IDXW)

    zeros128 = jnp.zeros((ZROWS, HD), jnp.float32)
    ones64 = jnp.ones((IDXW, HD), jnp.float32)

    deg2 = _sc_degree(cidx, ones64, zeros128)
    u0, dd, rinv = _tc_prep(deg2, x)

    us = [u0]
    u = u0
    for _ in range(4):
        v = _sc_hop(u.reshape(NC * N_NODES, HD), gidx, cidx, zeros128)
        u = _tc_glue(v, u, dd)
        us.append(u)

    Wr = W.reshape(7, D, D)
    br = b.reshape(7, 1, D)
    return _tc_dense(
        us[0], us[1], us[2], us[4], rinv,
        Wr, br,
        att_pre_low.reshape(D, D), att_pre_band.reshape(D, D),
        att_channel_low.reshape(D, D), att_channel_band.reshape(D, D),
        W_mlp.reshape(2, D, D), b_mlp.reshape(1, D),
    )


# speedup vs baseline: 12.3080x; 1.0019x over previous
import functools

import jax
import jax.numpy as jnp
from jax import lax
from jax.experimental import pallas as pl
from jax.experimental.pallas import tpu as pltpu
from jax.experimental.pallas import tpu_sc as plsc

N_NODES = 10000
N_EDGES = 160000
D = 256
HD = 128

NC = 2
NS = 16
IDXW = 64
EPAD = 163840
NPH = 2
ROWS_PH = 80
ROWS_SUB = NPH * ROWS_PH
ACC_N = 10112
ZROWS = ACC_N // NS
DRAIN = 624
DTAIL = N_NODES - NS * DRAIN

BLK = 400
GRID = N_NODES // BLK


def _sc_mesh():
    return plsc.VectorSubcoreMesh(core_axis_name="c", subcore_axis_name="s")


def _deg_body(cidx_hbm, ones_hbm, zeros_hbm, deg_hbm, acc, ci_v, ones_v, sem):
    c = lax.axis_index("c")
    s = lax.axis_index("s")
    pltpu.sync_copy(zeros_hbm, acc.at[pl.ds(s * ZROWS, ZROWS)])
    plsc.subcore_barrier()
    pltpu.sync_copy(ones_hbm, ones_v)
    pltpu.sync_copy(cidx_hbm.at[s, c], ci_v)

    @pl.loop(0, ROWS_PH)
    def _(j):
        pltpu.sync_copy(ones_v, acc.at[ci_v.at[j]], add=True)

    plsc.subcore_barrier()
    pltpu.sync_copy(acc.at[pl.ds(s * DRAIN, DRAIN)],
                    deg_hbm.at[c, pl.ds(s * DRAIN, DRAIN)])

    @pl.when(s == 0)
    def _():
        pltpu.sync_copy(acc.at[pl.ds(NS * DRAIN, DTAIL)],
                        deg_hbm.at[c, pl.ds(NS * DRAIN, DTAIL)])


def _sc_degree(cidx, ones64, zeros128):
    kern = pl.kernel(
        _deg_body,
        out_type=jax.ShapeDtypeStruct((NC, N_NODES, HD), jnp.float32),
        mesh=_sc_mesh(),
        scratch_types=[
            pltpu.VMEM_SHARED((ACC_N, HD), jnp.float32),
            pltpu.VMEM((ROWS_PH, IDXW), jnp.int32),
            pltpu.VMEM((IDXW, HD), jnp.float32),
            pltpu.SemaphoreType.DMA,
        ],
    )
    return kern(cidx, ones64, zeros128)


def _hop_body(u_hbm, gidx_hbm, cidx_hbm, zeros_hbm, v_hbm,
              acc, gi_v, ci_v, rows0, rows1, sem0, sem1):
    c = lax.axis_index("c")
    s = lax.axis_index("s")
    pltpu.sync_copy(zeros_hbm, acc.at[pl.ds(s * ZROWS, ZROWS)])
    plsc.subcore_barrier()
    for p in range(NPH):
        pltpu.sync_copy(gidx_hbm.at[c, s, p], gi_v)
        pltpu.sync_copy(cidx_hbm.at[s, p], ci_v)

        pltpu.async_copy(u_hbm.at[gi_v.at[0]], rows0, sem0)

        @pl.loop(0, ROWS_PH // 2)
        def _(jj):
            j = jj * 2
            pltpu.async_copy(u_hbm.at[gi_v.at[j + 1]], rows1, sem1)
            pltpu.make_async_copy(u_hbm.at[gi_v.at[j]], rows0, sem0).wait()
            pltpu.sync_copy(rows0, acc.at[ci_v.at[j]], add=True)

            @pl.when(jj < ROWS_PH // 2 - 1)
            def _():
                pltpu.async_copy(u_hbm.at[gi_v.at[j + 2]], rows0, sem0)

            pltpu.make_async_copy(u_hbm.at[gi_v.at[j + 1]], rows1, sem1).wait()
            pltpu.sync_copy(rows1, acc.at[ci_v.at[j + 1]], add=True)

    plsc.subcore_barrier()
    pltpu.sync_copy(acc.at[pl.ds(s * DRAIN, DRAIN)],
                    v_hbm.at[c, pl.ds(s * DRAIN, DRAIN)])

    @pl.when(s == 0)
    def _():
        pltpu.sync_copy(acc.at[pl.ds(NS * DRAIN, DTAIL)],
                        v_hbm.at[c, pl.ds(NS * DRAIN, DTAIL)])


def _sc_hop(u_flat, gidx, cidx, zeros128):
    kern = pl.kernel(
        _hop_body,
        out_type=jax.ShapeDtypeStruct((NC, N_NODES, HD), jnp.float32),
        mesh=_sc_mesh(),
        scratch_types=[
            pltpu.VMEM_SHARED((ACC_N, HD), jnp.float32),
            pltpu.VMEM((ROWS_PH, IDXW), jnp.int32),
            pltpu.VMEM((ROWS_PH, IDXW), jnp.int32),
            pltpu.VMEM((IDXW, HD), jnp.float32),
            pltpu.VMEM((IDXW, HD), jnp.float32),
            pltpu.SemaphoreType.DMA,
            pltpu.SemaphoreType.DMA,
        ],
    )
    return kern(u_flat, gidx, cidx, zeros128)


def _prep_body(degp_ref, x_ref, u0_ref, dd_ref, rinv_ref):
    deg = degp_ref[0, :, 0:1] + degp_ref[1, :, 0:1] + 1.0
    dinv = lax.rsqrt(deg)
    dd_ref[...] = jnp.broadcast_to(1.0 / deg, dd_ref.shape)
    rinv_ref[...] = jnp.broadcast_to(jnp.sqrt(deg), rinv_ref.shape)
    x = x_ref[...]
    u0_ref[0] = x[:, :HD] * dinv
    u0_ref[1] = x[:, HD:] * dinv


def _tc_prep(deg2, x):
    return pl.pallas_call(
        _prep_body,
        grid=(GRID,),
        in_specs=[
            pl.BlockSpec((NC, BLK, HD), lambda i: (0, i, 0)),
            pl.BlockSpec((BLK, D), lambda i: (i, 0)),
        ],
        out_specs=[
            pl.BlockSpec((NC, BLK, HD), lambda i: (0, i, 0)),
            pl.BlockSpec((BLK, 8), lambda i: (i, 0)),
            pl.BlockSpec((BLK, 8), lambda i: (i, 0)),
        ],
        out_shape=[
            jax.ShapeDtypeStruct((NC, N_NODES, HD), jnp.float32),
            jax.ShapeDtypeStruct((N_NODES, 8), jnp.float32),
            jax.ShapeDtypeStruct((N_NODES, 8), jnp.float32),
        ],
    )(deg2, x)


def _glue_body(v_ref, u_ref, dd_ref, un_ref):
    dd = dd_ref[:, 0:1][None]
    un_ref[...] = dd * (v_ref[...] + u_ref[...])


def _tc_glue(v, u, dd):
    return pl.pallas_call(
        _glue_body,
        grid=(GRID,),
        in_specs=[
            pl.BlockSpec((NC, BLK, HD), lambda i: (0, i, 0)),
            pl.BlockSpec((NC, BLK, HD), lambda i: (0, i, 0)),
            pl.BlockSpec((BLK, 8), lambda i: (i, 0)),
        ],
        out_specs=pl.BlockSpec((NC, BLK, HD), lambda i: (0, i, 0)),
        out_shape=jax.ShapeDtypeStruct((NC, N_NODES, HD), jnp.float32),
    )(v, u, dd)


def _elu(z):
    return jnp.where(z > 0, z, jnp.exp(jnp.minimum(z, 0.0)) - 1.0)


def _dot(a, b):
    return jnp.dot(a.astype(jnp.bfloat16), b.astype(jnp.bfloat16),
                   preferred_element_type=jnp.float32)


def _dense_body(u0_ref, u1_ref, u2_ref, u4_ref, rinv_ref,
                W_ref, b_ref, apl_ref, apb_ref, acl_ref, acb_ref,
                Wm_ref, bm_ref, out_ref):
    rinv = rinv_ref[:, 0:1]

    def xk(u_ref):
        return jnp.concatenate([u_ref[0], u_ref[1]], axis=-1) * rinv

    x0 = xk(u0_ref)
    x1 = xk(u1_ref)
    x2 = xk(u2_ref)
    x4 = xk(u4_ref)
    chans = (x0, x1, x2, x4, x0 - x1, x1 - x2, x2 - x4)
    feats = [_elu(_dot(chans[i], W_ref[i]) + b_ref[i]) for i in range(7)]

    def att(pre_feat, stack_feats, a_pre_ref, a_chan_ref):
        e_pre = _dot(pre_feat, a_pre_ref[...])
        es = [_dot(f, a_chan_ref[...]) + e_pre for f in stack_feats]
        m = jnp.maximum(jnp.maximum(es[0], es[1]), es[2])
        ws = [jnp.exp(e - m) for e in es]
        tot = ws[0] + ws[1] + ws[2]
        acc = ws[0] * stack_feats[0]
        acc += ws[1] * stack_feats[1]
        acc += ws[2] * stack_feats[2]
        return acc / tot

    x_low = att(feats[0], feats[1:4], apl_ref, acl_ref)
    x_band = att(feats[4], feats[4:7], apb_ref, acb_ref)
    out_ref[...] = (_dot(x_low, Wm_ref[0]) + _dot(x_band, Wm_ref[1])
                    + bm_ref[...])


def _tc_dense(u0, u1, u2, u4, rinv, W, b, apl, apb, acl, acb, Wm, bm):
    uspec = pl.BlockSpec((NC, BLK, HD), lambda i: (0, i, 0))
    full = lambda *shape: pl.BlockSpec(shape, lambda i: (0,) * len(shape))
    return pl.pallas_call(
        _dense_body,
        grid=(GRID,),
        in_specs=[
            uspec, uspec, uspec, uspec,
            pl.BlockSpec((BLK, 8), lambda i: (i, 0)),
            full(7, D, D),
            full(7, 1, D),
            full(D, D),
            full(D, D),
            full(D, D),
            full(D, D),
            full(2, D, D),
            full(1, D),
        ],
        out_specs=pl.BlockSpec((BLK, D), lambda i: (i, 0)),
        out_shape=jax.ShapeDtypeStruct((N_NODES, D), jnp.float32),
    )(u0, u1, u2, u4, rinv, W, b, apl, apb, acl, acb, Wm, bm)


def kernel(x, edge_index, W, b, att_pre_low, att_pre_band,
           att_channel_low, att_channel_band, W_mlp, b_mlp):
    row = edge_index[0].astype(jnp.int32)
    col = edge_index[1].astype(jnp.int32)

    npad = EPAD - N_EDGES
    pad_ids = jnp.arange(npad, dtype=jnp.int32)
    rowp = jnp.concatenate([row, (pad_ids * 13) % N_NODES])
    colp = jnp.concatenate([col, N_NODES + (pad_ids % 64)])

    gidx = jnp.stack([rowp, rowp + N_NODES]).reshape(NC, NS, NPH, ROWS_PH,
                                                     IDXW)
    cidx = colp.reshape(NS, NPH, ROWS_PH, IDXW)

    zeros128 = jnp.zeros((ZROWS, HD), jnp.float32)
    ones64 = jnp.ones((IDXW, HD), jnp.float32)

    deg2 = _sc_degree(cidx, ones64, zeros128)
    u0, dd, rinv = _tc_prep(deg2, x)

    us = [u0]
    u = u0
    for _ in range(4):
        v = _sc_hop(u.reshape(NC * N_NODES, HD), gidx, cidx, zeros128)
        u = _tc_glue(v, u, dd)
        us.append(u)

    Wr = W.reshape(7, D, D)
    br = b.reshape(7, 1, D)
    return _tc_dense(
        us[0], us[1], us[2], us[4], rinv,
        Wr, br,
        att_pre_low.reshape(D, D), att_pre_band.reshape(D, D),
        att_channel_low.reshape(D, D), att_channel_band.reshape(D, D),
        W_mlp.reshape(2, D, D), b_mlp.reshape(1, D),
    )
